# initial kernel scaffold (unmeasured)
import jax
import jax.numpy as jnp
from jax import lax
from jax.experimental import pallas as pl
from jax.experimental.pallas import tpu as pltpu


def kernel(
    x,
):
    def body(*refs):
        pass

    out_shape = jax.ShapeDtypeStruct(..., jnp.float32)
    return pl.pallas_call(body, out_shape=out_shape)(...)


# baseline (device time: 606042 ns/iter reference)
import jax
import jax.numpy as jnp
from jax import lax
from jax.experimental import pallas as pl
from jax.experimental.pallas import tpu as pltpu

N_Y = 4


def kernel(x):
    m_per, n = x.shape
    m_tot = N_Y * m_per

    def body(x_ref, out_ref, copy_sem, send_sems, recv_sems):
        my_x = lax.axis_index("x")
        my_y = lax.axis_index("y")
        my_z = lax.axis_index("z")
        left = (my_y - 1) % N_Y
        right = (my_y + 1) % N_Y

        barrier_sem = pltpu.get_barrier_semaphore()
        for nbr in (left, right):
            pl.semaphore_signal(
                barrier_sem,
                inc=1,
                device_id=(my_x, nbr, my_z),
                device_id_type=pl.DeviceIdType.MESH,
            )
        pl.semaphore_wait(barrier_sem, 2)

        local = pltpu.make_async_copy(
            x_ref, out_ref.at[pl.ds(my_y * m_per, m_per), :], copy_sem
        )
        local.start()
        local.wait()

        for h in range(N_Y - 1):
            origin = (my_y - h) % N_Y
            rdma = pltpu.make_async_remote_copy(
                src_ref=out_ref.at[pl.ds(origin * m_per, m_per), :],
                dst_ref=out_ref.at[pl.ds(origin * m_per, m_per), :],
                send_sem=send_sems.at[h],
                recv_sem=recv_sems.at[h],
                device_id=(my_x, right, my_z),
                device_id_type=pl.DeviceIdType.MESH,
            )
            rdma.start()
            rdma.wait()

    return pl.pallas_call(
        body,
        out_shape=jax.ShapeDtypeStruct((m_tot, n), x.dtype),
        in_specs=[pl.BlockSpec(memory_space=pl.ANY)],
        out_specs=pl.BlockSpec(memory_space=pl.ANY),
        scratch_shapes=[
            pltpu.SemaphoreType.DMA,
            pltpu.SemaphoreType.DMA((N_Y - 1,)),
            pltpu.SemaphoreType.DMA((N_Y - 1,)),
        ],
        compiler_params=pltpu.CompilerParams(collective_id=0),
    )(x)


# device time: 367290 ns/iter; 1.6500x vs baseline; 1.6500x over previous
import jax
import jax.numpy as jnp
from jax import lax
from jax.experimental import pallas as pl
from jax.experimental.pallas import tpu as pltpu

N_Y = 4
S = 4


def kernel(x):
    m_per, n = x.shape
    mh = m_per // 2
    ms = mh // S
    m_tot = N_Y * m_per
    T = (N_Y - 1) * S

    def body(x_ref, out_ref, copy_sem,
             sr_send, rl_recv, sl_send, rr_recv,
             xl_send, xl_recv, xr_send, xr_recv):
        my_x = lax.axis_index("x")
        my_y = lax.axis_index("y")
        my_z = lax.axis_index("z")
        partner = (1 - my_x, my_y, my_z)
        right_dev = (my_x, my_y + 1, my_z)
        left_dev = (my_x, my_y - 1, my_z)
        has_left = my_y > 0
        has_right = my_y < N_Y - 1

        barrier = pltpu.get_barrier_semaphore()
        pl.semaphore_signal(
            barrier, inc=1, device_id=partner,
            device_id_type=pl.DeviceIdType.MESH,
        )

        @pl.when(has_left)
        def _():
            pl.semaphore_signal(
                barrier, inc=1, device_id=left_dev,
                device_id_type=pl.DeviceIdType.MESH,
            )

        @pl.when(has_right)
        def _():
            pl.semaphore_signal(
                barrier, inc=1, device_id=right_dev,
                device_id_type=pl.DeviceIdType.MESH,
            )

        n_nbrs = (
            1 + has_left.astype(jnp.int32) + has_right.astype(jnp.int32)
        )
        pl.semaphore_wait(barrier, n_nbrs)

        local = pltpu.make_async_copy(
            x_ref, out_ref.at[pl.ds(my_y * m_per, m_per), :], copy_sem
        )
        local.start()

        def piece(chunk, half, j):
            return out_ref.at[pl.ds(chunk * m_per + half * mh + j * ms, ms), :]

        def own_piece(j):
            return x_ref.at[pl.ds(my_x * mh + j * ms, ms), :]

        def y_send(t, s, j, going_right):
            chunk = my_y - s if going_right else my_y + s
            src = own_piece(j) if s == 0 else piece(chunk, my_x, j)
            ssem, rsem = (sr_send, rl_recv) if going_right else (sl_send, rr_recv)
            return pltpu.make_async_remote_copy(
                src_ref=src,
                dst_ref=piece(chunk, my_x, j),
                send_sem=ssem.at[t],
                recv_sem=rsem.at[t],
                device_id=right_dev if going_right else left_dev,
                device_id_type=pl.DeviceIdType.MESH,
            )

        def x_fwd(t, s, j, from_left):
            chunk = my_y - 1 - s if from_left else my_y + 1 + s
            ssem, rsem = (xl_send, xl_recv) if from_left else (xr_send, xr_recv)
            return pltpu.make_async_remote_copy(
                src_ref=piece(chunk, my_x, j),
                dst_ref=piece(chunk, my_x, j),
                send_sem=ssem.at[t],
                recv_sem=rsem.at[t],
                device_id=partner,
                device_id_type=pl.DeviceIdType.MESH,
            )

        def x_in(t, s, j, from_left):
            chunk = my_y - 1 - s if from_left else my_y + 1 + s
            ssem, rsem = (xl_send, xl_recv) if from_left else (xr_send, xr_recv)
            return pltpu.make_async_remote_copy(
                src_ref=piece(chunk, 1 - my_x, j),
                dst_ref=piece(chunk, 1 - my_x, j),
                send_sem=ssem.at[t],
                recv_sem=rsem.at[t],
                device_id=partner,
                device_id_type=pl.DeviceIdType.MESH,
            )

        send_r = lambda s: has_right & (s <= my_y)
        send_l = lambda s: has_left & (s <= N_Y - 1 - my_y)
        recv_l = lambda s: has_left & (s <= my_y - 1)
        recv_r = lambda s: has_right & (s <= N_Y - 2 - my_y)

        for t in range(T):
            s, j = divmod(t, S)

            @pl.when(send_r(s))
            def _(t=t, s=s, j=j):
                y_send(t, s, j, going_right=True).start()

            @pl.when(send_l(s))
            def _(t=t, s=s, j=j):
                y_send(t, s, j, going_right=False).start()

            @pl.when(recv_l(s))
            def _(t=t, s=s, j=j):
                y_send(t, s, j, going_right=True).wait_recv()
                x_fwd(t, s, j, from_left=True).start()

            @pl.when(recv_r(s))
            def _(t=t, s=s, j=j):
                y_send(t, s, j, going_right=False).wait_recv()
                x_fwd(t, s, j, from_left=False).start()

        local.wait()

        for t in range(T):
            s, j = divmod(t, S)

            @pl.when(recv_l(s))
            def _(t=t, s=s, j=j):
                x_in(t, s, j, from_left=True).wait_recv()
                x_fwd(t, s, j, from_left=True).wait_send()

            @pl.when(recv_r(s))
            def _(t=t, s=s, j=j):
                x_in(t, s, j, from_left=False).wait_recv()
                x_fwd(t, s, j, from_left=False).wait_send()

            @pl.when(send_r(s))
            def _(t=t, s=s, j=j):
                y_send(t, s, j, going_right=True).wait_send()

            @pl.when(send_l(s))
            def _(t=t, s=s, j=j):
                y_send(t, s, j, going_right=False).wait_send()

    return pl.pallas_call(
        body,
        out_shape=jax.ShapeDtypeStruct((m_tot, n), x.dtype),
        in_specs=[pl.BlockSpec(memory_space=pl.ANY)],
        out_specs=pl.BlockSpec(memory_space=pl.ANY),
        scratch_shapes=[
            pltpu.SemaphoreType.DMA,
            pltpu.SemaphoreType.DMA((T,)),
            pltpu.SemaphoreType.DMA((T,)),
            pltpu.SemaphoreType.DMA((T,)),
            pltpu.SemaphoreType.DMA((T,)),
            pltpu.SemaphoreType.DMA((T,)),
            pltpu.SemaphoreType.DMA((T,)),
            pltpu.SemaphoreType.DMA((T,)),
            pltpu.SemaphoreType.DMA((T,)),
        ],
        compiler_params=pltpu.CompilerParams(collective_id=0),
    )(x)
